# baseline (speedup 1.0000x reference)
import functools
import jax
import jax.numpy as jnp
from jax import lax
from jax.experimental import pallas as pl
from jax.experimental.pallas import tpu as pltpu, tpu_sc as plsc

_B = 32
_V = 1000000
_CHUNK = 20000
_NCHUNK = _V // _CHUNK
_NVEC = _CHUNK // 16
_UN = 10
_LO = -16.0
_S1 = 256.0
_NB1 = 8192
_S2 = 1048576.0
_NB2 = 4096
_TOP_P = 0.9

_mesh = plsc.VectorSubcoreMesh(core_axis_name="c", subcore_axis_name="s")


def _suffix_search(h_ref, nvec, target):
    io = lax.iota(jnp.int32, 16)

    def step(jj, carry):
        cs, found, kstar, s_at, h_at = carry
        j = nvec - 1 - jj
        v = h_ref[pl.ds(j * 16, 16)]
        w = lax.rev(plsc.cumsum(lax.rev(v, (0,))), (0,))
        s = w + cs
        mask = s > target
        cnt = jnp.max(plsc.all_reduce_population_count(mask))
        has = (cnt > 0).astype(jnp.int32)
        i0 = cnt - 1
        pick = io == i0
        sv = jnp.sum(jnp.where(pick, s, 0.0))
        hv = jnp.sum(jnp.where(pick, v, 0.0))
        take = (has == 1) & (found == 0)
        return (jnp.max(s),
                found | has,
                jnp.where(take, j * 16 + i0, kstar),
                jnp.where(take, sv, s_at),
                jnp.where(take, hv, h_at))

    init = (jnp.float32(0.0), jnp.int32(0), jnp.int32(0),
            jnp.float32(0.0), jnp.float32(0.0))
    _, _, kstar, s_at, h_at = lax.fori_loop(0, nvec, step, init)
    return kstar, s_at, h_at


@functools.partial(
    pl.kernel, mesh=_mesh,
    out_type=jax.ShapeDtypeStruct((_B * _V,), jnp.float32),
    scratch_types=[
        pltpu.VMEM((_CHUNK,), jnp.float32),
        pltpu.VMEM((_CHUNK,), jnp.float32),
        pltpu.VMEM((_NB1,), jnp.float32),
        pltpu.VMEM((_NB2,), jnp.float32),
    ],
    compiler_params=pltpu.CompilerParams(needs_layout_passes=False),
)
def _sc_topp(x_hbm, out_hbm, inbuf, outbuf, h1, h2):
    wid = lax.axis_index("s") * 2 + lax.axis_index("c")
    base = wid * _V
    zero16 = jnp.zeros((16,), jnp.float32)

    def z1(j, _):
        h1[pl.ds(j * 16, 16)] = zero16
        return 0

    lax.fori_loop(0, _NB1 // 16, z1, 0)

    def z2(j, _):
        h2[pl.ds(j * 16, 16)] = zero16
        return 0

    lax.fori_loop(0, _NB2 // 16, z2, 0)

    def chunk1(c, zacc):
        pltpu.sync_copy(x_hbm.at[pl.ds(base + c * _CHUNK, _CHUNK)], inbuf)

        def vstep(k, za):
            for u in range(_UN):
                l = inbuf[pl.ds((k * _UN + u) * 16, 16)]
                e = jnp.exp(l)
                b = jnp.clip((l - _LO) * _S1, 0.0, _NB1 - 1).astype(jnp.int32)
                plsc.addupdate_scatter(h1, [b], e)
                za = za + e
            return za

        return lax.fori_loop(0, _NVEC // _UN, vstep, zacc)

    zacc = lax.fori_loop(0, _NCHUNK, chunk1, zero16)
    z = jnp.sum(zacc)
    target = jnp.float32(_TOP_P) * z

    bstar, s_at, h_at = _suffix_search(h1, _NB1 // 16, target)
    s_hi = s_at - h_at
    edge = _LO + bstar.astype(jnp.float32) * jnp.float32(1.0 / _S1)

    def chunk2(c, _):
        pltpu.sync_copy(x_hbm.at[pl.ds(base + c * _CHUNK, _CHUNK)], inbuf)

        def vstep(k, _2):
            for u in range(_UN):
                l = inbuf[pl.ds((k * _UN + u) * 16, 16)]
                e = jnp.exp(l)
                b = jnp.clip((l - _LO) * _S1, 0.0, _NB1 - 1).astype(jnp.int32)
                sub = jnp.clip((l - edge) * _S2, 0.0, _NB2 - 1).astype(jnp.int32)
                plsc.addupdate_scatter(h2, [sub], e, mask=b == bstar)
            return 0

        return lax.fori_loop(0, _NVEC // _UN, vstep, 0)

    lax.fori_loop(0, _NCHUNK, chunk2, 0)

    k2, s2_at, _ = _suffix_search(h2, _NB2 // 16, target - s_hi)
    inv = (zero16 + jnp.float32(1.0)) / (zero16 + (s_hi + s2_at))

    def chunk3(c, _):
        pltpu.sync_copy(x_hbm.at[pl.ds(base + c * _CHUNK, _CHUNK)], inbuf)

        def vstep(k, _2):
            for u in range(_UN):
                sl = pl.ds((k * _UN + u) * 16, 16)
                l = inbuf[sl]
                e = jnp.exp(l)
                b = jnp.clip((l - _LO) * _S1, 0.0, _NB1 - 1).astype(jnp.int32)
                sub = jnp.clip((l - edge) * _S2, 0.0, _NB2 - 1).astype(jnp.int32)
                keep = (b > bstar) | ((b == bstar) & (sub >= k2))
                outbuf[sl] = jnp.where(keep, e * inv, 0.0)
            return 0

        lax.fori_loop(0, _NVEC // _UN, vstep, 0)
        pltpu.sync_copy(outbuf, out_hbm.at[pl.ds(base + c * _CHUNK, _CHUNK)])
        return 0

    lax.fori_loop(0, _NCHUNK, chunk3, 0)


def kernel(logits):
    b, v = logits.shape
    assert b == _B and v == _V
    out = _sc_topp(logits.reshape(-1))
    return out.reshape(b, v)


# speedup vs baseline: 1.2731x; 1.2731x over previous
import functools
import jax
import jax.numpy as jnp
from jax import lax
from jax.experimental import pallas as pl
from jax.experimental.pallas import tpu as pltpu, tpu_sc as plsc

_B = 32
_V = 1000000
_CHUNK = 20000
_NCHUNK = _V // _CHUNK
_NVEC = _CHUNK // 16
_UN = 10
_LO = -16.0
_S1 = 256.0
_NB1 = 8192
_S2 = 1048576.0
_NB2 = 4096
_TOP_P = 0.9

_mesh = plsc.VectorSubcoreMesh(core_axis_name="c", subcore_axis_name="s")


def _suffix_search(h_ref, nvec, target):
    io = lax.iota(jnp.int32, 16)

    def step(jj, carry):
        cs, found, kstar, s_at, h_at = carry
        j = nvec - 1 - jj
        v = h_ref[pl.ds(j * 16, 16)]
        w = lax.rev(plsc.cumsum(lax.rev(v, (0,))), (0,))
        s = w + cs
        mask = s > target
        cnt = jnp.max(plsc.all_reduce_population_count(mask))
        has = (cnt > 0).astype(jnp.int32)
        i0 = cnt - 1
        pick = io == i0
        sv = jnp.sum(jnp.where(pick, s, 0.0))
        hv = jnp.sum(jnp.where(pick, v, 0.0))
        take = (has == 1) & (found == 0)
        return (jnp.max(s),
                found | has,
                jnp.where(take, j * 16 + i0, kstar),
                jnp.where(take, sv, s_at),
                jnp.where(take, hv, h_at))

    init = (jnp.float32(0.0), jnp.int32(0), jnp.int32(0),
            jnp.float32(0.0), jnp.float32(0.0))
    _, _, kstar, s_at, h_at = lax.fori_loop(0, nvec, step, init)
    return kstar, s_at, h_at


@functools.partial(
    pl.kernel, mesh=_mesh,
    out_type=jax.ShapeDtypeStruct((_B * _V,), jnp.float32),
    scratch_types=[
        pltpu.VMEM((_CHUNK,), jnp.float32),
        pltpu.VMEM((_CHUNK,), jnp.float32),
        pltpu.VMEM((_NB1,), jnp.float32),
        pltpu.VMEM((_NB2,), jnp.float32),
    ],
    compiler_params=pltpu.CompilerParams(needs_layout_passes=False),
)
def _sc_topp(x_hbm, out_hbm, inbuf, outbuf, h1, h2):
    wid = lax.axis_index("s") * 2 + lax.axis_index("c")
    base = wid * _V
    zero16 = jnp.zeros((16,), jnp.float32)

    def z1(j, _):
        h1[pl.ds(j * 16, 16)] = zero16
        return 0

    lax.fori_loop(0, _NB1 // 16, z1, 0)

    def z2(j, _):
        h2[pl.ds(j * 16, 16)] = zero16
        return 0

    lax.fori_loop(0, _NB2 // 16, z2, 0)

    def chunk1(c, zacc):
        pltpu.sync_copy(x_hbm.at[pl.ds(base + c * _CHUNK, _CHUNK)], inbuf)

        def vstep(i, za):
            l = inbuf[pl.ds(i, 16)]
            e = jnp.exp(l)
            b = jnp.clip((l - _LO) * _S1, 0.0, _NB1 - 1).astype(jnp.int32)
            plsc.addupdate_scatter(h1, [b], e)
            return za + e

        return plsc.parallel_loop(0, _CHUNK, 16, unroll=_UN, carry=zacc)(vstep)

    zacc = lax.fori_loop(0, _NCHUNK, chunk1, zero16)
    z = jnp.sum(zacc)
    target = jnp.float32(_TOP_P) * z

    bstar, s_at, h_at = _suffix_search(h1, _NB1 // 16, target)
    s_hi = s_at - h_at
    edge = _LO + bstar.astype(jnp.float32) * jnp.float32(1.0 / _S1)

    def chunk2(c, _):
        pltpu.sync_copy(x_hbm.at[pl.ds(base + c * _CHUNK, _CHUNK)], inbuf)

        def vstep(i):
            l = inbuf[pl.ds(i, 16)]
            e = jnp.exp(l)
            b = jnp.clip((l - _LO) * _S1, 0.0, _NB1 - 1).astype(jnp.int32)
            sub = jnp.clip((l - edge) * _S2, 0.0, _NB2 - 1).astype(jnp.int32)
            plsc.addupdate_scatter(h2, [sub], e, mask=b == bstar)

        plsc.parallel_loop(0, _CHUNK, 16, unroll=_UN)(vstep)
        return 0

    lax.fori_loop(0, _NCHUNK, chunk2, 0)

    k2, s2_at, _ = _suffix_search(h2, _NB2 // 16, target - s_hi)
    inv = (zero16 + jnp.float32(1.0)) / (zero16 + (s_hi + s2_at))

    def chunk3(c, _):
        pltpu.sync_copy(x_hbm.at[pl.ds(base + c * _CHUNK, _CHUNK)], inbuf)

        def vstep(i):
            sl = pl.ds(i, 16)
            l = inbuf[sl]
            e = jnp.exp(l)
            b = jnp.clip((l - _LO) * _S1, 0.0, _NB1 - 1).astype(jnp.int32)
            sub = jnp.clip((l - edge) * _S2, 0.0, _NB2 - 1).astype(jnp.int32)
            keep = (b > bstar) | ((b == bstar) & (sub >= k2))
            outbuf[sl] = jnp.where(keep, e * inv, 0.0)

        plsc.parallel_loop(0, _CHUNK, 16, unroll=_UN)(vstep)
        pltpu.sync_copy(outbuf, out_hbm.at[pl.ds(base + c * _CHUNK, _CHUNK)])
        return 0

    lax.fori_loop(0, _NCHUNK, chunk3, 0)


def kernel(logits):
    b, v = logits.shape
    assert b == _B and v == _V
    out = _sc_topp(logits.reshape(-1))
    return out.reshape(b, v)


# speedup vs baseline: 1.3219x; 1.0383x over previous
import functools
import jax
import jax.numpy as jnp
from jax import lax
from jax.experimental import pallas as pl
from jax.experimental.pallas import tpu as pltpu, tpu_sc as plsc

_B = 32
_V = 1000000
_CHUNK = 20000
_NCHUNK = _V // _CHUNK
_NVEC = _CHUNK // 16
_UN = 10
_LO = -16.0
_S1 = 256.0
_NB1 = 8192
_S2 = 1048576.0
_NB2 = 4096
_TOP_P = 0.9

_mesh = plsc.VectorSubcoreMesh(core_axis_name="c", subcore_axis_name="s")


def _suffix_search(h_ref, nvec, target):
    io = lax.iota(jnp.int32, 16)

    def step(jj, carry):
        cs, found, kstar, s_at, h_at = carry
        j = nvec - 1 - jj
        v = h_ref[pl.ds(j * 16, 16)]
        w = lax.rev(plsc.cumsum(lax.rev(v, (0,))), (0,))
        s = w + cs
        mask = s > target
        cnt = jnp.max(plsc.all_reduce_population_count(mask))
        has = (cnt > 0).astype(jnp.int32)
        i0 = cnt - 1
        pick = io == i0
        sv = jnp.sum(jnp.where(pick, s, 0.0))
        hv = jnp.sum(jnp.where(pick, v, 0.0))
        take = (has == 1) & (found == 0)
        return (jnp.max(s),
                found | has,
                jnp.where(take, j * 16 + i0, kstar),
                jnp.where(take, sv, s_at),
                jnp.where(take, hv, h_at))

    init = (jnp.float32(0.0), jnp.int32(0), jnp.int32(0),
            jnp.float32(0.0), jnp.float32(0.0))
    _, _, kstar, s_at, h_at = lax.fori_loop(0, nvec, step, init)
    return kstar, s_at, h_at


@functools.partial(
    pl.kernel, mesh=_mesh,
    out_type=jax.ShapeDtypeStruct((_B * _V,), jnp.float32),
    scratch_types=[
        pltpu.VMEM((_CHUNK,), jnp.float32),
        pltpu.VMEM((_CHUNK,), jnp.float32),
        pltpu.VMEM((_CHUNK,), jnp.float32),
        pltpu.VMEM((_CHUNK,), jnp.float32),
        pltpu.VMEM((_NB1,), jnp.float32),
        pltpu.VMEM((_NB2,), jnp.float32),
        pltpu.SemaphoreType.DMA,
        pltpu.SemaphoreType.DMA,
        pltpu.SemaphoreType.DMA,
        pltpu.SemaphoreType.DMA,
    ],
    compiler_params=pltpu.CompilerParams(needs_layout_passes=False),
)
def _sc_topp(x_hbm, out_hbm, in_a, in_b, out_a, out_b, h1, h2,
             sem_a, sem_b, osem_a, osem_b):
    wid = lax.axis_index("s") * 2 + lax.axis_index("c")
    base = wid * _V
    zero16 = jnp.zeros((16,), jnp.float32)

    def _in_slice(c):
        return x_hbm.at[pl.ds(base + c * _CHUNK, _CHUNK)]

    def _start_in(c, buf, sem):
        pltpu.async_copy(_in_slice(c), buf, sem)

    def _wait_in(buf, sem):
        pltpu.make_async_copy(_in_slice(0), buf, sem).wait()

    def _pipe(body, carry0):
        _start_in(0, in_a, sem_a)

        def pair(cp, carry):
            c0 = cp * 2
            _start_in(c0 + 1, in_b, sem_b)
            _wait_in(in_a, sem_a)
            carry = body(in_a, 0, c0, carry)

            @pl.when(c0 + 2 < _NCHUNK)
            def _():
                _start_in(c0 + 2, in_a, sem_a)

            _wait_in(in_b, sem_b)
            return body(in_b, 1, c0 + 1, carry)

        return lax.fori_loop(0, _NCHUNK // 2, pair, carry0)

    def z1(j, _):
        h1[pl.ds(j * 16, 16)] = zero16
        return 0

    lax.fori_loop(0, _NB1 // 16, z1, 0)

    def z2(j, _):
        h2[pl.ds(j * 16, 16)] = zero16
        return 0

    lax.fori_loop(0, _NB2 // 16, z2, 0)

    def body1(buf, parity, c, zacc):
        def vstep(i, za):
            l = buf[pl.ds(i, 16)]
            e = jnp.exp(l)
            b = jnp.clip((l - _LO) * _S1, 0.0, _NB1 - 1).astype(jnp.int32)
            plsc.addupdate_scatter(h1, [b], e)
            return za + e

        return plsc.parallel_loop(0, _CHUNK, 16, unroll=_UN, carry=zacc)(vstep)

    zacc = _pipe(body1, zero16)
    z = jnp.sum(zacc)
    target = jnp.float32(_TOP_P) * z

    bstar, s_at, h_at = _suffix_search(h1, _NB1 // 16, target)
    s_hi = s_at - h_at
    edge = _LO + bstar.astype(jnp.float32) * jnp.float32(1.0 / _S1)

    def body2(buf, parity, c, carry):
        def vstep(i):
            l = buf[pl.ds(i, 16)]
            e = jnp.exp(l)
            b = jnp.clip((l - _LO) * _S1, 0.0, _NB1 - 1).astype(jnp.int32)
            sub = jnp.clip((l - edge) * _S2, 0.0, _NB2 - 1).astype(jnp.int32)
            plsc.addupdate_scatter(h2, [sub], e, mask=b == bstar)

        plsc.parallel_loop(0, _CHUNK, 16, unroll=_UN)(vstep)
        return carry

    _pipe(body2, jnp.int32(0))

    k2, s2_at, _ = _suffix_search(h2, _NB2 // 16, target - s_hi)
    inv = (zero16 + jnp.float32(1.0)) / (zero16 + (s_hi + s2_at))

    def _out_slice(c):
        return out_hbm.at[pl.ds(base + c * _CHUNK, _CHUNK)]

    def body3(buf, parity, c, carry):
        obuf = out_a if parity == 0 else out_b
        osem = osem_a if parity == 0 else osem_b

        @pl.when(c >= 2)
        def _():
            pltpu.make_async_copy(_in_slice(0), obuf, osem).wait()

        def vstep(i):
            sl = pl.ds(i, 16)
            l = buf[sl]
            e = jnp.exp(l)
            b = jnp.clip((l - _LO) * _S1, 0.0, _NB1 - 1).astype(jnp.int32)
            sub = jnp.clip((l - edge) * _S2, 0.0, _NB2 - 1).astype(jnp.int32)
            keep = (b > bstar) | ((b == bstar) & (sub >= k2))
            obuf[sl] = jnp.where(keep, e * inv, 0.0)

        plsc.parallel_loop(0, _CHUNK, 16, unroll=_UN)(vstep)
        pltpu.async_copy(obuf, _out_slice(c), osem)
        return carry

    _pipe(body3, jnp.int32(0))
    pltpu.make_async_copy(_in_slice(0), out_a, osem_a).wait()
    pltpu.make_async_copy(_in_slice(0), out_b, osem_b).wait()


def kernel(logits):
    b, v = logits.shape
    assert b == _B and v == _V
    out = _sc_topp(logits.reshape(-1))
    return out.reshape(b, v)


# speedup vs baseline: 8.6954x; 6.5782x over previous
import jax
import jax.numpy as jnp
from jax.experimental import pallas as pl

_R = 1000
_C = 1000
_CH = 8
_U = 5
_NIT = _R // (_CH * _U)
_U2 = 25
_NIT2 = _R // (_CH * _U2)
_TOP_P = 0.9
_NSAMP = 16
_NITER = 8


def _row_body(x_ref, o_ref):
    zeros = jnp.zeros((_CH, _C), jnp.float32)

    def p1(i, carry):
        acc, mx = carry
        for u in range(_U2):
            sl = pl.ds((i * _U2 + u) * _CH, _CH)
            e = jnp.exp(x_ref[0, sl, :])
            o_ref[0, sl, :] = e
            acc = acc + e
            mx = jnp.maximum(mx, e)
        return acc, mx

    acc, mxv = jax.lax.fori_loop(0, _NIT2, p1, (zeros, zeros))
    z = jnp.sum(acc)
    maxe = jnp.max(mxv)
    target = jnp.float32(_TOP_P) * z
    hi0 = maxe * jnp.float32(1.001) + jnp.float32(1.0)

    e0 = o_ref[0, pl.ds(0, _CH), :]
    targ_s = jnp.float32(_TOP_P) * jnp.sum(e0)

    def astep(_, carry):
        lo, hi = carry
        t = 0.5 * (lo + hi)
        m = jnp.sum(jnp.where(e0 >= t, e0, 0.0))
        big = m > targ_s
        return jnp.where(big, t, lo), jnp.where(big, hi, t)

    alo, ahi = jax.lax.fori_loop(0, _NSAMP, astep, (jnp.float32(0.0), hi0))
    t_hat = 0.5 * (alo + ahi)
    b_lo = t_hat * jnp.float32(0.88)
    b_hi = t_hat * jnp.float32(1.14)

    def bstep(i, carry):
        lo, hi, zk = carry
        w = hi - lo
        q1 = lo + 0.25 * w
        q2 = lo + 0.5 * w
        q3 = lo + 0.75 * w
        t1 = jnp.where(i == 0, b_lo, q1)
        t2 = jnp.where(i == 0, t_hat, q2)
        t3 = jnp.where(i == 0, b_hi, q3)
        t1 = jnp.where((t1 > lo) & (t1 < hi), t1, q1)
        t2 = jnp.where((t2 > lo) & (t2 < hi), t2, q2)
        t3 = jnp.where((t3 > lo) & (t3 < hi), t3, q3)
        a, b = jnp.minimum(t1, t2), jnp.maximum(t1, t2)
        t1 = jnp.minimum(a, t3)
        c = jnp.maximum(a, t3)
        t2 = jnp.minimum(b, c)
        t3 = jnp.maximum(b, c)

        def mstep(j, accs):
            a1, a2, a3 = accs
            for u in range(_U):
                e = o_ref[0, pl.ds((j * _U + u) * _CH, _CH), :]
                a1 = a1 + jnp.where(e >= t1, e, 0.0)
                a2 = a2 + jnp.where(e >= t2, e, 0.0)
                a3 = a3 + jnp.where(e >= t3, e, 0.0)
            return a1, a2, a3

        a1, a2, a3 = jax.lax.fori_loop(0, _NIT, mstep, (zeros, zeros, zeros))
        m1, m2, m3 = jnp.sum(a1), jnp.sum(a2), jnp.sum(a3)
        b1, b2, b3 = m1 > target, m2 > target, m3 > target
        lo2 = jnp.where(b3, t3, jnp.where(b2, t2, jnp.where(b1, t1, lo)))
        zk2 = jnp.where(b3, m3, jnp.where(b2, m2, jnp.where(b1, m1, zk)))
        hi2 = jnp.where(~b1, t1, jnp.where(~b2, t2, jnp.where(~b3, t3, hi)))
        return lo2, hi2, zk2

    lo, _, zk = jax.lax.fori_loop(
        0, _NITER, bstep, (jnp.float32(0.0), hi0, z))

    inv = jnp.float32(1.0) / zk

    def p3(i, _):
        for u in range(_U2):
            sl = pl.ds((i * _U2 + u) * _CH, _CH)
            e = o_ref[0, sl, :]
            o_ref[0, sl, :] = jnp.where(e >= lo, e * inv, 0.0)
        return 0

    jax.lax.fori_loop(0, _NIT2, p3, 0)


def kernel(logits):
    b, v = logits.shape
    assert v == _R * _C
    x3 = logits.reshape(b, _R, _C)
    out = pl.pallas_call(
        _row_body,
        grid=(b,),
        in_specs=[pl.BlockSpec((1, _R, _C), lambda i: (i, 0, 0))],
        out_specs=pl.BlockSpec((1, _R, _C), lambda i: (i, 0, 0)),
        out_shape=jax.ShapeDtypeStruct((b, _R, _C), jnp.float32),
    )(x3)
    return out.reshape(b, v)


# speedup vs baseline: 9.1431x; 1.0515x over previous
import jax
import jax.numpy as jnp
from jax.experimental import pallas as pl

_R = 1000
_C = 1000
_CH = 8
_U = 5
_NIT = _R // (_CH * _U)
_U2 = 25
_NIT2 = _R // (_CH * _U2)
_TOP_P = 0.9
_NSAMP = 16
_NITER = 7


def _row_body(x_ref, o_ref):
    zeros = jnp.zeros((_CH, _C), jnp.float32)

    def p1(i, carry):
        acc, mx = carry
        for u in range(_U2):
            sl = pl.ds((i * _U2 + u) * _CH, _CH)
            e = jnp.exp(x_ref[0, sl, :])
            o_ref[0, sl, :] = e
            acc = acc + e
            mx = jnp.maximum(mx, e)
        return acc, mx

    acc, mxv = jax.lax.fori_loop(0, _NIT2, p1, (zeros, zeros))
    z = jnp.sum(acc)
    maxe = jnp.max(mxv)
    target = jnp.float32(_TOP_P) * z
    hi0 = maxe * jnp.float32(1.001) + jnp.float32(1.0)

    e0 = o_ref[0, pl.ds(0, _CH), :]
    targ_s = jnp.float32(_TOP_P) * jnp.sum(e0)

    def astep(_, carry):
        lo, hi = carry
        t = 0.5 * (lo + hi)
        m = jnp.sum(jnp.where(e0 >= t, e0, 0.0))
        big = m > targ_s
        return jnp.where(big, t, lo), jnp.where(big, hi, t)

    alo, ahi = jax.lax.fori_loop(0, _NSAMP, astep, (jnp.float32(0.0), hi0))
    t_hat = 0.5 * (alo + ahi)
    b_lo = t_hat * jnp.float32(0.88)
    b_hi = t_hat * jnp.float32(1.14)

    def bstep(i, carry):
        lo, hi, zk = carry
        w = hi - lo
        q1 = lo + 0.25 * w
        q2 = lo + 0.5 * w
        q3 = lo + 0.75 * w
        t1 = jnp.where(i == 0, b_lo, q1)
        t2 = jnp.where(i == 0, t_hat, q2)
        t3 = jnp.where(i == 0, b_hi, q3)
        t1 = jnp.where((t1 > lo) & (t1 < hi), t1, q1)
        t2 = jnp.where((t2 > lo) & (t2 < hi), t2, q2)
        t3 = jnp.where((t3 > lo) & (t3 < hi), t3, q3)
        a, b = jnp.minimum(t1, t2), jnp.maximum(t1, t2)
        t1 = jnp.minimum(a, t3)
        c = jnp.maximum(a, t3)
        t2 = jnp.minimum(b, c)
        t3 = jnp.maximum(b, c)

        def mstep(j, accs):
            a1, a2, a3 = accs
            for u in range(_U):
                e = o_ref[0, pl.ds((j * _U + u) * _CH, _CH), :]
                a1 = a1 + jnp.where(e >= t1, e, 0.0)
                a2 = a2 + jnp.where(e >= t2, e, 0.0)
                a3 = a3 + jnp.where(e >= t3, e, 0.0)
            return a1, a2, a3

        a1, a2, a3 = jax.lax.fori_loop(0, _NIT, mstep, (zeros, zeros, zeros))
        m1, m2, m3 = jnp.sum(a1), jnp.sum(a2), jnp.sum(a3)
        b1, b2, b3 = m1 > target, m2 > target, m3 > target
        lo2 = jnp.where(b3, t3, jnp.where(b2, t2, jnp.where(b1, t1, lo)))
        zk2 = jnp.where(b3, m3, jnp.where(b2, m2, jnp.where(b1, m1, zk)))
        hi2 = jnp.where(~b1, t1, jnp.where(~b2, t2, jnp.where(~b3, t3, hi)))
        return lo2, hi2, zk2

    lo, _, zk = jax.lax.fori_loop(
        0, _NITER, bstep, (jnp.float32(0.0), hi0, z))

    inv = jnp.float32(1.0) / zk

    def p3(i, _):
        for u in range(_U2):
            sl = pl.ds((i * _U2 + u) * _CH, _CH)
            e = o_ref[0, sl, :]
            o_ref[0, sl, :] = jnp.where(e >= lo, e * inv, 0.0)
        return 0

    jax.lax.fori_loop(0, _NIT2, p3, 0)


def kernel(logits):
    b, v = logits.shape
    assert v == _R * _C
    x3 = logits.reshape(b, _R, _C)
    out = pl.pallas_call(
        _row_body,
        grid=(b,),
        in_specs=[pl.BlockSpec((1, _R, _C), lambda i: (i, 0, 0))],
        out_specs=pl.BlockSpec((1, _R, _C), lambda i: (i, 0, 0)),
        out_shape=jax.ShapeDtypeStruct((b, _R, _C), jnp.float32),
    )(x3)
    return out.reshape(b, v)
